# baseline (speedup 1.0000x reference)
import jax
import jax.numpy as jnp
from jax import lax
from jax.experimental import pallas as pl
from jax.experimental.pallas import tpu as pltpu
from jax.experimental.pallas import tpu_sc as plsc

NC = 2
NS = 16
LANES = 16

B = 4
L = 4096
D = 128
NW = NC * NS
POS_W = L // NW
CHUNK = 64
NCHUNK = B * POS_W // CHUNK
NBUF = 3


def _emb_body(x_hbm, wt_hbm, pos_hbm, out_hbm,
              idx_v, pos_v, w0_v, w1_v, w2_v,
              sem_i0, sem_i1, sem_i2, sem_i3, sem_p,
              sem_g0, sem_g1, sem_g2, sem_s0, sem_s1, sem_s2):
    wid = lax.axis_index("s") * NC + lax.axis_index("c")
    p0 = wid * POS_W

    word_bufs = (w0_v, w1_v, w2_v)
    isems = (sem_i0, sem_i1, sem_i2, sem_i3)
    gsems = (sem_g0, sem_g1, sem_g2)
    ssems = (sem_s0, sem_s1, sem_s2)

    pos_cp = pltpu.async_copy(pos_hbm.at[pl.ds(p0, POS_W)], pos_v, sem_p)
    idx_cps = [
        pltpu.async_copy(x_hbm.at[b, pl.ds(p0, POS_W)],
                         idx_v.at[pl.ds(b * POS_W, POS_W)], isems[b])
        for b in range(B)
    ]
    idx_done = [False] * B

    def fire_gather(c):
        b = c * CHUNK // POS_W
        if not idx_done[b]:
            idx_cps[b].wait()
            idx_done[b] = True
        return pltpu.async_copy(
            wt_hbm.at[idx_v.at[pl.ds(c * CHUNK, CHUNK)]],
            word_bufs[c % NBUF], gsems[c % NBUF])

    gathers = [None] * NCHUNK
    stores = [None] * NCHUNK
    gathers[0] = fire_gather(0)
    gathers[1] = fire_gather(1)

    pos_cp.wait()

    for c in range(NCHUNK):
        buf = c % NBUF
        gathers[c].wait()
        if c + 2 < NCHUNK:
            if stores[c - 1] is not None:
                stores[c - 1].wait()
            gathers[c + 2] = fire_gather(c + 2)

        word_v = word_bufs[buf]
        b = c * CHUNK // POS_W
        poff = (c * CHUNK) % POS_W

        def row(r, rc):
            for j in range(D // LANES):
                sl = pl.ds(j * LANES, LANES)
                word_v[r, sl] = word_v[r, sl] + pos_v[poff + r, sl]
            return rc

        lax.fori_loop(0, CHUNK, row, 0)
        stores[c] = pltpu.async_copy(
            word_v, out_hbm.at[b, pl.ds(p0 + poff, CHUNK)], ssems[buf])

    stores[NCHUNK - 3].wait()
    stores[NCHUNK - 2].wait()
    stores[NCHUNK - 1].wait()


_emb = pl.kernel(
    _emb_body,
    out_type=jax.ShapeDtypeStruct((B, L, D), jnp.float32),
    mesh=plsc.VectorSubcoreMesh(
        core_axis_name="c", subcore_axis_name="s", num_cores=NC, num_subcores=NS
    ),
    scratch_types=[
        pltpu.VMEM((B * POS_W,), jnp.int32),
        pltpu.VMEM((POS_W, D), jnp.float32),
        pltpu.VMEM((CHUNK, D), jnp.float32),
        pltpu.VMEM((CHUNK, D), jnp.float32),
        pltpu.VMEM((CHUNK, D), jnp.float32),
    ] + [pltpu.SemaphoreType.DMA] * 11,
)


def kernel(x, word_table, pos_table):
    return _emb(x.astype(jnp.int32), word_table, pos_table)


# speedup vs baseline: 1.0376x; 1.0376x over previous
import jax
import jax.numpy as jnp
from jax import lax
from jax.experimental import pallas as pl
from jax.experimental.pallas import tpu as pltpu
from jax.experimental.pallas import tpu_sc as plsc

NC = 2
NS = 16
LANES = 16

B = 4
L = 4096
D = 128
NW = NC * NS
POS_W = L // NW
NBUF = 3


def _emb_body(x_hbm, wt_hbm, pos_hbm, out_hbm,
              idx_v, pos_v, w0_v, w1_v, w2_v,
              sem_i0, sem_i1, sem_i2, sem_i3, sem_p,
              sem_g0, sem_g1, sem_g2, sem_s0, sem_s1, sem_s2):
    wid = lax.axis_index("s") * NC + lax.axis_index("c")
    p0 = wid * POS_W

    word_bufs = (w0_v, w1_v, w2_v)
    isems = (sem_i0, sem_i1, sem_i2, sem_i3)
    gsems = (sem_g0, sem_g1, sem_g2)
    ssems = (sem_s0, sem_s1, sem_s2)

    pos_cp = pltpu.async_copy(pos_hbm.at[pl.ds(p0, POS_W)], pos_v, sem_p)
    idx_cps = [
        pltpu.async_copy(x_hbm.at[b, pl.ds(p0, POS_W)],
                         idx_v.at[pl.ds(b * POS_W, POS_W)], isems[b])
        for b in range(B)
    ]

    def fire_gather(b):
        idx_cps[b].wait()
        return pltpu.async_copy(
            wt_hbm.at[idx_v.at[pl.ds(b * POS_W, POS_W)]],
            word_bufs[b % NBUF], gsems[b % NBUF])

    gathers = [None] * B
    stores = [None] * B
    gathers[0] = fire_gather(0)
    gathers[1] = fire_gather(1)

    pos_cp.wait()

    for b in range(B):
        buf = b % NBUF
        gathers[b].wait()
        if b + 2 < B:
            if stores[b - 1] is not None:
                stores[b - 1].wait()
            gathers[b + 2] = fire_gather(b + 2)

        word_v = word_bufs[buf]

        def row(r, rc):
            for j in range(D // LANES):
                sl = pl.ds(j * LANES, LANES)
                word_v[r, sl] = word_v[r, sl] + pos_v[r, sl]
            return rc

        if b < B - 1:
            lax.fori_loop(0, POS_W, row, 0)
            stores[b] = pltpu.async_copy(
                word_v, out_hbm.at[b, pl.ds(p0, POS_W)], ssems[buf])
        else:
            half = POS_W // 2
            lax.fori_loop(0, half, row, 0)
            first_half = pltpu.async_copy(
                word_v.at[pl.ds(0, half)],
                out_hbm.at[b, pl.ds(p0, half)], sem_i0)
            lax.fori_loop(half, POS_W, row, 0)
            stores[b] = pltpu.async_copy(
                word_v.at[pl.ds(half, half)],
                out_hbm.at[b, pl.ds(p0 + half, half)], ssems[buf])
            first_half.wait()

    stores[B - 3].wait()
    stores[B - 2].wait()
    stores[B - 1].wait()


_emb = pl.kernel(
    _emb_body,
    out_type=jax.ShapeDtypeStruct((B, L, D), jnp.float32),
    mesh=plsc.VectorSubcoreMesh(
        core_axis_name="c", subcore_axis_name="s", num_cores=NC, num_subcores=NS
    ),
    scratch_types=[
        pltpu.VMEM((B * POS_W,), jnp.int32),
        pltpu.VMEM((POS_W, D), jnp.float32),
        pltpu.VMEM((POS_W, D), jnp.float32),
        pltpu.VMEM((POS_W, D), jnp.float32),
        pltpu.VMEM((POS_W, D), jnp.float32),
    ] + [pltpu.SemaphoreType.DMA] * 11,
)


def kernel(x, word_table, pos_table):
    return _emb(x.astype(jnp.int32), word_table, pos_table)
